# baseline (speedup 1.0000x reference)
import functools

import jax
import jax.numpy as jnp
from jax import lax
from jax.experimental import pallas as pl
from jax.experimental.pallas import tpu as pltpu
from jax.experimental.pallas import tpu_sc as plsc

_NC = 2
_NS = 16
_NW = _NC * _NS


def _sc_add(xf, pef, rows, S, D):
    rpw = rows // _NW
    TILE = 16
    n_tiles = rpw // TILE
    wpb = S // rpw

    @functools.partial(
        pl.kernel,
        out_type=jax.ShapeDtypeStruct((rows * D,), jnp.float32),
        mesh=plsc.VectorSubcoreMesh(core_axis_name="c", subcore_axis_name="s"),
        scratch_types=[
            pltpu.VMEM((TILE * D,), jnp.float32),
            pltpu.VMEM((TILE * D,), jnp.float32),
        ],
    )
    def k(x_hbm, pe_hbm, out_hbm, xbuf, pebuf):
        wid = lax.axis_index("s") * _NC + lax.axis_index("c")
        base = wid * (rpw * D)
        pe_base = lax.rem(wid, wpb) * (rpw * D)

        def tile_body(t, _):
            off = t * (TILE * D)
            pltpu.sync_copy(x_hbm.at[pl.ds(base + off, TILE * D)], xbuf)
            pltpu.sync_copy(pe_hbm.at[pl.ds(pe_base + off, TILE * D)], pebuf)

            def add16(i, carry):
                o = i * 16
                xbuf[pl.ds(o, 16)] = xbuf[pl.ds(o, 16)] + pebuf[pl.ds(o, 16)]
                return carry

            lax.fori_loop(0, (TILE * D) // 16, add16, 0)
            pltpu.sync_copy(xbuf, out_hbm.at[pl.ds(base + off, TILE * D)])
            return _

        lax.fori_loop(0, n_tiles, tile_body, 0)

    return k(xf, pef)


def kernel(x, pe_table):
    B, S, D = x.shape
    rows = B * S
    out = _sc_add(x.reshape(rows * D), pe_table.reshape(S * D), rows, S, D)
    return out.reshape(B, S, D)


# speedup vs baseline: 1.5523x; 1.5523x over previous
import functools

import jax
import jax.numpy as jnp
from jax import lax
from jax.experimental import pallas as pl
from jax.experimental.pallas import tpu as pltpu
from jax.experimental.pallas import tpu_sc as plsc

_NC = 2
_NS = 16
_NW = _NC * _NS


def _sc_add(xf, pef, rows, S, D):
    rpw = rows // _NW
    TILE = 16
    n_tiles = rpw // TILE
    wpb = S // rpw

    UNROLL = 8

    @functools.partial(
        pl.kernel,
        out_type=jax.ShapeDtypeStruct((rows * D,), jnp.float32),
        mesh=plsc.VectorSubcoreMesh(core_axis_name="c", subcore_axis_name="s"),
        scratch_types=[
            pltpu.VMEM((2, TILE * D), jnp.float32),
            pltpu.VMEM((2, TILE * D), jnp.float32),
            pltpu.VMEM((2, TILE * D), jnp.float32),
            pltpu.SemaphoreType.DMA,
            pltpu.SemaphoreType.DMA,
        ],
    )
    def k(x_hbm, pe_hbm, out_hbm, xbuf, pebuf, obuf, insem, outsem):
        wid = lax.axis_index("s") * _NC + lax.axis_index("c")
        base = wid * (rpw * D)
        pe_base = lax.rem(wid, wpb) * (rpw * D)

        def in_copies(t, slot):
            off = t * (TILE * D)
            cx = pltpu.make_async_copy(
                x_hbm.at[pl.ds(base + off, TILE * D)], xbuf.at[slot], insem)
            cp = pltpu.make_async_copy(
                pe_hbm.at[pl.ds(pe_base + off, TILE * D)], pebuf.at[slot], insem)
            return cx, cp

        def out_copy(t, slot):
            off = t * (TILE * D)
            return pltpu.make_async_copy(
                obuf.at[slot], out_hbm.at[pl.ds(base + off, TILE * D)], outsem)

        for s in (0, 1):
            cx, cp = in_copies(s, s)
            cx.start()
            cp.start()

        def tile_body(t, slot):
            @pl.when(t >= 2)
            def _drain():
                out_copy(t - 2, slot).wait()

            cx, cp = in_copies(t, slot)
            cx.wait()
            cp.wait()

            def addv(i, carry):
                o = i * (16 * UNROLL)
                for u in range(UNROLL):
                    q = o + u * 16
                    obuf[slot, pl.ds(q, 16)] = (
                        xbuf[slot, pl.ds(q, 16)] + pebuf[slot, pl.ds(q, 16)])
                return carry

            lax.fori_loop(0, (TILE * D) // (16 * UNROLL), addv, 0)
            out_copy(t, slot).start()

            @pl.when(t + 2 < n_tiles)
            def _refill():
                nx, np_ = in_copies(t + 2, slot)
                nx.start()
                np_.start()

        def pair_body(p, carry):
            tile_body(2 * p, 0)
            tile_body(2 * p + 1, 1)
            return carry

        lax.fori_loop(0, n_tiles // 2, pair_body, 0)

        out_copy(n_tiles - 2, 0).wait()
        out_copy(n_tiles - 1, 1).wait()

    return k(xf, pef)


def kernel(x, pe_table):
    B, S, D = x.shape
    rows = B * S
    out = _sc_add(x.reshape(rows * D), pe_table.reshape(S * D), rows, S, D)
    return out.reshape(B, S, D)


# speedup vs baseline: 2.1222x; 1.3671x over previous
import functools

import jax
import jax.numpy as jnp
from jax import lax
from jax.experimental import pallas as pl
from jax.experimental.pallas import tpu as pltpu
from jax.experimental.pallas import tpu_sc as plsc

_NC = 2
_NS = 16
_NW = _NC * _NS

_SC_ROWS = 1024
_TC_BLK = 512


def _tc_body(x_ref, pe_ref, o_ref):
    i = pl.program_id(0)
    S = pe_ref.shape[0]
    base = lax.rem(i * _TC_BLK, S)
    o_ref[...] = x_ref[...] + pe_ref[pl.ds(base, _TC_BLK), :]


def _tc_add(xf, pe_table, n_rows):
    D = xf.shape[1]
    S = pe_table.shape[0]
    return pl.pallas_call(
        _tc_body,
        out_shape=jax.ShapeDtypeStruct((n_rows, D), xf.dtype),
        grid=(n_rows // _TC_BLK,),
        in_specs=[
            pl.BlockSpec((_TC_BLK, D), lambda i: (i, 0)),
            pl.BlockSpec((S, D), lambda i: (0, 0)),
        ],
        out_specs=pl.BlockSpec((_TC_BLK, D), lambda i: (i, 0)),
    )(xf, pe_table)


def _sc_add(xf, pef, n_rows, D, x_row0, pe_row0):
    rpw = n_rows // _NW
    TILE = 16
    n_tiles = rpw // TILE
    UNROLL = 8

    @functools.partial(
        pl.kernel,
        out_type=jax.ShapeDtypeStruct((n_rows * D,), jnp.float32),
        mesh=plsc.VectorSubcoreMesh(core_axis_name="c", subcore_axis_name="s"),
        scratch_types=[
            pltpu.VMEM((2, TILE * D), jnp.float32),
            pltpu.VMEM((2, TILE * D), jnp.float32),
            pltpu.VMEM((2, TILE * D), jnp.float32),
            pltpu.SemaphoreType.DMA,
            pltpu.SemaphoreType.DMA,
        ],
    )
    def k(x_hbm, pe_hbm, out_hbm, xbuf, pebuf, obuf, insem, outsem):
        wid = lax.axis_index("s") * _NC + lax.axis_index("c")
        base = wid * (rpw * D)

        def in_copies(t, slot):
            off = base + t * (TILE * D)
            cx = pltpu.make_async_copy(
                x_hbm.at[pl.ds(x_row0 * D + off, TILE * D)], xbuf.at[slot],
                insem)
            cp = pltpu.make_async_copy(
                pe_hbm.at[pl.ds(pe_row0 * D + off, TILE * D)], pebuf.at[slot],
                insem)
            return cx, cp

        def out_copy(t, slot):
            off = base + t * (TILE * D)
            return pltpu.make_async_copy(
                obuf.at[slot], out_hbm.at[pl.ds(off, TILE * D)], outsem)

        for s in (0, 1):
            cx, cp = in_copies(s, s)
            cx.start()
            cp.start()

        def tile_body(t, slot):
            @pl.when(t >= 2)
            def _drain():
                out_copy(t - 2, slot).wait()

            cx, cp = in_copies(t, slot)
            cx.wait()
            cp.wait()

            def addv(i, carry):
                o = i * (16 * UNROLL)
                for u in range(UNROLL):
                    q = o + u * 16
                    obuf[slot, pl.ds(q, 16)] = (
                        xbuf[slot, pl.ds(q, 16)] + pebuf[slot, pl.ds(q, 16)])
                return carry

            lax.fori_loop(0, (TILE * D) // (16 * UNROLL), addv, 0)
            out_copy(t, slot).start()

            @pl.when(t + 2 < n_tiles)
            def _refill():
                nx, np_ = in_copies(t + 2, slot)
                nx.start()
                np_.start()

        def pair_body(p, carry):
            tile_body(2 * p, 0)
            tile_body(2 * p + 1, 1)
            return carry

        lax.fori_loop(0, n_tiles // 2, pair_body, 0)
        out_copy(n_tiles - 2, 0).wait()
        out_copy(n_tiles - 1, 1).wait()

    return k(xf, pef)


def kernel(x, pe_table):
    B, S, D = x.shape
    rows = B * S
    K = _SC_ROWS
    xf = x.reshape(rows, D)
    tc_out = _tc_add(xf, pe_table, rows - K)
    sc_out = _sc_add(
        xf.reshape(rows * D), pe_table.reshape(S * D),
        K, D, rows - K, S - K)
    out = jnp.concatenate([tc_out, sc_out.reshape(K, D)], axis=0)
    return out.reshape(B, S, D)


# speedup vs baseline: 8.1553x; 3.8429x over previous
import functools

import jax
import jax.numpy as jnp
from jax import lax
from jax.experimental import pallas as pl
from jax.experimental.pallas import tpu as pltpu
from jax.experimental.pallas import tpu_sc as plsc

_NC = 2
_NS = 16
_NW = _NC * _NS

_SC_ROWS = 0
_TC_BLK = 512


def _tc_body(x_ref, pe_ref, o_ref):
    i = pl.program_id(0)
    S = pe_ref.shape[0]
    base = lax.rem(i * _TC_BLK, S)
    o_ref[...] = x_ref[...] + pe_ref[pl.ds(base, _TC_BLK), :]


def _tc_add(xf, pe_table, n_rows):
    D = xf.shape[1]
    S = pe_table.shape[0]
    return pl.pallas_call(
        _tc_body,
        out_shape=jax.ShapeDtypeStruct((n_rows, D), xf.dtype),
        grid=(n_rows // _TC_BLK,),
        in_specs=[
            pl.BlockSpec((_TC_BLK, D), lambda i: (i, 0)),
            pl.BlockSpec((S, D), lambda i: (0, 0)),
        ],
        out_specs=pl.BlockSpec((_TC_BLK, D), lambda i: (i, 0)),
    )(xf, pe_table)


def _sc_add(xf, pef, n_rows, D, x_row0, pe_row0):
    rpw = n_rows // _NW
    TILE = 16
    n_tiles = rpw // TILE
    UNROLL = 8

    @functools.partial(
        pl.kernel,
        out_type=jax.ShapeDtypeStruct((n_rows * D,), jnp.float32),
        mesh=plsc.VectorSubcoreMesh(core_axis_name="c", subcore_axis_name="s"),
        scratch_types=[
            pltpu.VMEM((2, TILE * D), jnp.float32),
            pltpu.VMEM((2, TILE * D), jnp.float32),
            pltpu.VMEM((2, TILE * D), jnp.float32),
            pltpu.SemaphoreType.DMA,
            pltpu.SemaphoreType.DMA,
        ],
    )
    def k(x_hbm, pe_hbm, out_hbm, xbuf, pebuf, obuf, insem, outsem):
        wid = lax.axis_index("s") * _NC + lax.axis_index("c")
        base = wid * (rpw * D)

        def in_copies(t, slot):
            off = base + t * (TILE * D)
            cx = pltpu.make_async_copy(
                x_hbm.at[pl.ds(x_row0 * D + off, TILE * D)], xbuf.at[slot],
                insem)
            cp = pltpu.make_async_copy(
                pe_hbm.at[pl.ds(pe_row0 * D + off, TILE * D)], pebuf.at[slot],
                insem)
            return cx, cp

        def out_copy(t, slot):
            off = base + t * (TILE * D)
            return pltpu.make_async_copy(
                obuf.at[slot], out_hbm.at[pl.ds(off, TILE * D)], outsem)

        for s in (0, 1):
            cx, cp = in_copies(s, s)
            cx.start()
            cp.start()

        def tile_body(t, slot):
            @pl.when(t >= 2)
            def _drain():
                out_copy(t - 2, slot).wait()

            cx, cp = in_copies(t, slot)
            cx.wait()
            cp.wait()

            def addv(i, carry):
                o = i * (16 * UNROLL)
                for u in range(UNROLL):
                    q = o + u * 16
                    obuf[slot, pl.ds(q, 16)] = (
                        xbuf[slot, pl.ds(q, 16)] + pebuf[slot, pl.ds(q, 16)])
                return carry

            lax.fori_loop(0, (TILE * D) // (16 * UNROLL), addv, 0)
            out_copy(t, slot).start()

            @pl.when(t + 2 < n_tiles)
            def _refill():
                nx, np_ = in_copies(t + 2, slot)
                nx.start()
                np_.start()

        def pair_body(p, carry):
            tile_body(2 * p, 0)
            tile_body(2 * p + 1, 1)
            return carry

        lax.fori_loop(0, n_tiles // 2, pair_body, 0)
        out_copy(n_tiles - 2, 0).wait()
        out_copy(n_tiles - 1, 1).wait()

    return k(xf, pef)


def kernel(x, pe_table):
    B, S, D = x.shape
    rows = B * S
    xf = x.reshape(rows, D)
    return _tc_add(xf, pe_table, rows).reshape(B, S, D)
